# baseline (speedup 1.0000x reference)
import jax
import jax.numpy as jnp
from jax import lax
from jax.experimental import pallas as pl
from jax.experimental.pallas import tpu as pltpu

_R, _C = 256, 128
_N = _R * _C
_BR = 8
_BLK = _BR * _C
_NB = _R // _BR
_UNROLL = 8
_HUGE = 1e35
_TINY = 1e-8


def _sort_body(x_ref, off_ref, out_x_ref, out_i_ref, idxw_ref):
    ii = (lax.broadcasted_iota(jnp.int32, (_BR, _C), 0) * _C
          + lax.broadcasted_iota(jnp.int32, (_BR, _C), 1))

    def roll2(a, j):
        if j < _C:
            axis, s, size = 1, j, _C
        else:
            axis, s, size = 0, j // _C, _BR
        if 2 * s == size:
            r = jnp.roll(a, s, axis=axis)
            return r, r, True
        return jnp.roll(a, -s, axis=axis), jnp.roll(a, s, axis=axis), False

    def local_substage(x, idx, j, m, hh, tt, sgn_b):
        xm, xp, whole = roll2(x, j)
        im, ip, _ = roll2(idx, j)
        if whole:
            pv, pi = xm, im
        else:
            pv = jnp.where(m, xm, xp)
            pi = jnp.where(m, im, ip)
        q = (x - pv) * hh + (idx - pi) * tt
        if sgn_b is not None:
            q = q * sgn_b
        take_p = q > 0.0
        return jnp.where(take_p, pv, x), jnp.where(take_p, pi, idx)

    subs_a = [(1 << js, 1 << ks)
              for ks in range(1, 11) for js in range(ks - 1, -1, -1)]

    def run_local_multi(bs, xs, idxs, substages):
        for j, k in substages:
            m = (ii & j) == 0
            if k <= 512:
                wm = m == ((ii & k) == 0)
                sgns = [None] * len(bs)
            else:
                wm = m
                sgns = [jnp.where((b & (k // _BLK)) == 0, jnp.float32(1.0),
                                  jnp.float32(-1.0)) for b in bs]
            hh = jnp.where(wm, jnp.float32(_HUGE), jnp.float32(-_HUGE))
            tt = jnp.where(wm, jnp.float32(_TINY), jnp.float32(-_TINY))
            for t in range(len(xs)):
                xs[t], idxs[t] = local_substage(
                    xs[t], idxs[t], j, m, hh, tt, sgns[t])
        return xs, idxs

    def body_a(it, carry):
        bs = [it * _UNROLL + u for u in range(_UNROLL)]
        starts = [pl.multiple_of(b * _BR, _BR) for b in bs]
        xs = [x_ref[pl.ds(s, _BR), :] for s in starts]
        idxs = [(ii + b * _BLK).astype(jnp.float32) for b in bs]
        xs, idxs = run_local_multi(bs, xs, idxs, subs_a)
        for t, s in enumerate(starts):
            out_x_ref[pl.ds(s, _BR), :] = xs[t]
            idxw_ref[pl.ds(s, _BR), :] = idxs[t]
        return carry

    lax.fori_loop(0, _NB // _UNROLL, body_a, 0)

    for ks in range(11, 16):
        k = 1 << ks
        for d in (1 << t for t in range(ks - 11, -1, -1)):
            for p in range(_NB // 2):
                blo = (p // d) * 2 * d + (p % d)
                bhi = blo + d
                up = (blo & (k // _BLK)) == 0
                slo = pl.ds(blo * _BR, _BR)
                shi = pl.ds(bhi * _BR, _BR)
                xa = out_x_ref[slo, :]
                ia = idxw_ref[slo, :]
                xb = out_x_ref[shi, :]
                ib = idxw_ref[shi, :]
                q = (xa - xb) * _HUGE + (ia - ib) * _TINY
                swap = (q > 0.0) if up else (q < 0.0)
                out_x_ref[slo, :] = jnp.where(swap, xb, xa)
                idxw_ref[slo, :] = jnp.where(swap, ib, ia)
                out_x_ref[shi, :] = jnp.where(swap, xa, xb)
                idxw_ref[shi, :] = jnp.where(swap, ia, ib)
        subs_local = [(1 << js, k) for js in range(9, -1, -1)]

        def body_b(it, carry, subs_local=subs_local):
            bs = [it * _UNROLL + u for u in range(_UNROLL)]
            starts = [pl.multiple_of(b * _BR, _BR) for b in bs]
            xs = [out_x_ref[pl.ds(s, _BR), :] for s in starts]
            idxs = [idxw_ref[pl.ds(s, _BR), :] for s in starts]
            xs, idxs = run_local_multi(bs, xs, idxs, subs_local)
            for t, s in enumerate(starts):
                out_x_ref[pl.ds(s, _BR), :] = xs[t]
                idxw_ref[pl.ds(s, _BR), :] = idxs[t]
            return carry

        lax.fori_loop(0, _NB // _UNROLL, body_b, 0)

    out_x_ref[...] = out_x_ref[...] - off_ref[...]
    out_i_ref[...] = idxw_ref[...].astype(jnp.int32)


def kernel(input, node2graph):
    step = jnp.max(input, axis=0) - jnp.min(input, axis=0) + 1.0
    offset = node2graph.astype(input.dtype) * step
    x = input + offset
    out_x, out_i = pl.pallas_call(
        _sort_body,
        out_shape=(
            jax.ShapeDtypeStruct((_R, _C), jnp.float32),
            jax.ShapeDtypeStruct((_R, _C), jnp.int32),
        ),
        scratch_shapes=[pltpu.VMEM((_R, _C), jnp.float32)],
    )(x.reshape(_R, _C), offset.reshape(_R, _C))
    return out_x.reshape(_N), out_i.reshape(_N)


# speedup vs baseline: 1.4785x; 1.4785x over previous
import jax
import jax.numpy as jnp
from jax import lax
from jax.experimental import pallas as pl

_R, _C = 256, 128
_N = _R * _C
_HUGE = 1e35
_TINY = 1e-8
_LOG2N = 15


def _sort_body(x_ref, out_x_ref, out_i_ref):
    rr = lax.broadcasted_iota(jnp.int32, (_R, _C), 0)
    cc = lax.broadcasted_iota(jnp.int32, (_R, _C), 1)
    ii = cc * _R + rr
    H = jnp.float32(_HUGE)
    T = jnp.float32(_TINY)

    x = x_ref[...]
    idx = ii.astype(jnp.float32)

    def roll2(a, j):
        if j <= _C:
            axis, d, size = 0, j, _R
        else:
            axis, d, size = 1, j // _R, _C
        if 2 * d == size:
            r = jnp.roll(a, d, axis=axis)
            return r, r, True
        return jnp.roll(a, -d, axis=axis), jnp.roll(a, d, axis=axis), False

    for ks in range(1, _LOG2N + 1):
        k = 1 << ks
        for js in range(ks - 1, -1, -1):
            j = 1 << js
            if j <= _C:
                m = (rr & j) == 0
            else:
                m = (cc & (j // _R)) == 0
            if k <= _C:
                up = (rr & k) == 0
            else:
                up = (cc & (k // _R)) == 0
            wm = m == up
            hh = jnp.where(wm, H, -H)
            tt = jnp.where(wm, T, -T)
            xm, xp, whole = roll2(x, j)
            im, ip, _ = roll2(idx, j)
            if whole:
                pv, pi = xm, im
            else:
                pv = jnp.where(m, xm, xp)
                pi = jnp.where(m, im, ip)
            q = (x - pv) * hh + (idx - pi) * tt
            take_p = q > 0.0
            x = jnp.where(take_p, pv, x)
            idx = jnp.where(take_p, pi, idx)

    out_x_ref[...] = x
    out_i_ref[...] = idx.astype(jnp.int32)


def kernel(input, node2graph):
    step = jnp.max(input, axis=0) - jnp.min(input, axis=0) + 1.0
    offset = node2graph.astype(input.dtype) * step
    x = input + offset
    x_cm = x.reshape(_C, _R).T
    out_x, out_i = pl.pallas_call(
        _sort_body,
        out_shape=(
            jax.ShapeDtypeStruct((_R, _C), jnp.float32),
            jax.ShapeDtypeStruct((_R, _C), jnp.int32),
        ),
    )(x_cm)
    sorted_x = out_x.T.reshape(_N) - offset
    index = out_i.T.reshape(_N)
    return sorted_x, index


# speedup vs baseline: 1.4794x; 1.0006x over previous
import jax
import jax.numpy as jnp
from jax import lax
from jax.experimental import pallas as pl

_R, _C = 256, 128
_N = _R * _C
_HUGE = 1e35
_TINY = 1e-8
_LOG2N = 15


def _sort_body(x_ref, out_x_ref, out_i_ref):
    rr = lax.broadcasted_iota(jnp.int32, (_R, _C), 0)
    cc = lax.broadcasted_iota(jnp.int32, (_R, _C), 1)
    ii = cc * _R + rr
    H = jnp.float32(_HUGE)
    T = jnp.float32(_TINY)

    x = x_ref[...]
    idx = ii.astype(jnp.float32)

    def bitmask(j):
        if j <= _C:
            return (rr & j) == 0
        return (cc & (j // _R)) == 0

    masks = {1 << s: bitmask(1 << s) for s in range(_LOG2N + 1)}
    upmasks = masks

    def roll2(a, j):
        if j <= _C:
            axis, d, size = 0, j, _R
        else:
            axis, d, size = 1, j // _R, _C
        if 2 * d == size:
            r = jnp.roll(a, d, axis=axis)
            return r, r, True
        return jnp.roll(a, -d, axis=axis), jnp.roll(a, d, axis=axis), False

    for ks in range(1, _LOG2N + 1):
        k = 1 << ks
        for js in range(ks - 1, -1, -1):
            j = 1 << js
            m = masks[j]
            up = upmasks[k]
            wm = m == up
            hh = jnp.where(wm, H, -H)
            tt = jnp.where(wm, T, -T)
            xm, xp, whole = roll2(x, j)
            im, ip, _ = roll2(idx, j)
            if whole:
                pv, pi = xm, im
            else:
                pv = jnp.where(m, xm, xp)
                pi = jnp.where(m, im, ip)
            q = (x - pv) * hh + (idx - pi) * tt
            take_p = q > 0.0
            x = jnp.where(take_p, pv, x)
            idx = jnp.where(take_p, pi, idx)

    out_x_ref[...] = x
    out_i_ref[...] = idx.astype(jnp.int32)


def kernel(input, node2graph):
    step = jnp.max(input, axis=0) - jnp.min(input, axis=0) + 1.0
    offset = node2graph.astype(input.dtype) * step
    x = input + offset
    x_cm = x.reshape(_C, _R).T
    out_x, out_i = pl.pallas_call(
        _sort_body,
        out_shape=(
            jax.ShapeDtypeStruct((_R, _C), jnp.float32),
            jax.ShapeDtypeStruct((_R, _C), jnp.int32),
        ),
    )(x_cm)
    sorted_x = out_x.T.reshape(_N) - offset
    index = out_i.T.reshape(_N)
    return sorted_x, index


# speedup vs baseline: 1.5332x; 1.0363x over previous
import jax
import jax.numpy as jnp
from jax import lax
from jax.experimental import pallas as pl

_R, _C = 256, 128
_N = _R * _C
_HUGE = 1e35
_TINY = 1e-8
_LOG2N = 15


def _sort_body(x_ref, off_ref, out_x_ref, out_i_ref):
    rr = lax.broadcasted_iota(jnp.int32, (_R, _C), 0)
    cc = lax.broadcasted_iota(jnp.int32, (_R, _C), 1)
    ii = cc * _R + rr
    H = jnp.float32(_HUGE)
    T = jnp.float32(_TINY)

    x = x_ref[...]
    idx = (rr * _C + cc).astype(jnp.float32)

    def bitmask(j):
        if j <= _C:
            return (rr & j) == 0
        return (cc & (j // _R)) == 0

    masks = {1 << s: bitmask(1 << s) for s in range(_LOG2N + 1)}
    upmasks = masks

    def roll2(a, j):
        if j <= _C:
            axis, d, size = 0, j, _R
        else:
            axis, d, size = 1, j // _R, _C
        if 2 * d == size:
            r = jnp.roll(a, d, axis=axis)
            return r, r, True
        return jnp.roll(a, -d, axis=axis), jnp.roll(a, d, axis=axis), False

    for ks in range(1, _LOG2N + 1):
        k = 1 << ks
        for js in range(ks - 1, -1, -1):
            j = 1 << js
            m = masks[j]
            up = upmasks[k]
            wm = m == up
            hh = jnp.where(wm, H, -H)
            tt = jnp.where(wm, T, -T)
            xm, xp, whole = roll2(x, j)
            im, ip, _ = roll2(idx, j)
            if whole:
                pv, pi = xm, im
            else:
                pv = jnp.where(m, xm, xp)
                pi = jnp.where(m, im, ip)
            q = (x - pv) * hh + (idx - pi) * tt
            take_p = q > 0.0
            x = jnp.where(take_p, pv, x)
            idx = jnp.where(take_p, pi, idx)

    out_x_ref[...] = x.T - off_ref[...]
    out_i_ref[...] = idx.T.astype(jnp.int32)


def kernel(input, node2graph):
    step = jnp.max(input, axis=0) - jnp.min(input, axis=0) + 1.0
    offset = node2graph.astype(input.dtype) * step
    x = input + offset
    out_x, out_i = pl.pallas_call(
        _sort_body,
        out_shape=(
            jax.ShapeDtypeStruct((_C, _R), jnp.float32),
            jax.ShapeDtypeStruct((_C, _R), jnp.int32),
        ),
    )(x.reshape(_R, _C), offset.reshape(_C, _R))
    return out_x.reshape(_N), out_i.reshape(_N)


# speedup vs baseline: 1.5640x; 1.0201x over previous
import jax
import jax.numpy as jnp
from jax import lax
from jax.experimental import pallas as pl

_R, _C = 256, 128
_N = _R * _C
_HUGE = 1e35
_TINY = 1e-8
_LOG2N = 15


def _sort_body(x_ref, out_x_ref, out_i_ref):
    rr = lax.broadcasted_iota(jnp.int32, (_R, _C), 0)
    cc = lax.broadcasted_iota(jnp.int32, (_R, _C), 1)
    ii = cc * _R + rr
    H = jnp.float32(_HUGE)
    T = jnp.float32(_TINY)

    x = x_ref[...]
    idx = (rr * _C + cc).astype(jnp.float32)

    def bitmask(j):
        if j <= _C:
            return (rr & j) == 0
        return (cc & (j // _R)) == 0

    masks = {1 << s: bitmask(1 << s) for s in range(_LOG2N + 1)}
    upmasks = masks

    def roll2(a, j):
        if j <= _C:
            axis, d, size = 0, j, _R
        else:
            axis, d, size = 1, j // _R, _C
        if 2 * d == size:
            r = jnp.roll(a, d, axis=axis)
            return r, r, True
        return jnp.roll(a, -d, axis=axis), jnp.roll(a, d, axis=axis), False

    for ks in range(1, _LOG2N + 1):
        k = 1 << ks
        for js in range(ks - 1, -1, -1):
            j = 1 << js
            m = masks[j]
            up = upmasks[k]
            wm = m == up
            hh = jnp.where(wm, H, -H)
            tt = jnp.where(wm, T, -T)
            xm, xp, whole = roll2(x, j)
            im, ip, _ = roll2(idx, j)
            if whole:
                pv, pi = xm, im
            else:
                pv = jnp.where(m, xm, xp)
                pi = jnp.where(m, im, ip)
            q = (x - pv) * hh + (idx - pi) * tt
            take_p = q > 0.0
            x = jnp.where(take_p, pv, x)
            idx = jnp.where(take_p, pi, idx)

    out_x_ref[...] = x.T
    out_i_ref[...] = idx.T.astype(jnp.int32)


def kernel(input, node2graph):
    step = jnp.max(input, axis=0) - jnp.min(input, axis=0) + 1.0
    offset = node2graph.astype(input.dtype) * step
    x = input + offset
    out_x, out_i = pl.pallas_call(
        _sort_body,
        out_shape=(
            jax.ShapeDtypeStruct((_C, _R), jnp.float32),
            jax.ShapeDtypeStruct((_C, _R), jnp.int32),
        ),
    )(x.reshape(_R, _C))
    sorted_x = out_x.reshape(_N) - offset
    index = out_i.reshape(_N)
    return sorted_x, index
